# baseline (speedup 1.0000x reference)
import functools

import jax
import jax.numpy as jnp
from jax import lax
from jax.experimental import pallas as pl
from jax.experimental.pallas import tpu as pltpu
from jax.experimental.pallas import tpu_sc as plsc

_NUM_CORES = 2
_NUM_SUBCORES = 16
_NW = _NUM_CORES * _NUM_SUBCORES
_LANES = 16

_BT = 128
_NTILES = 4
_NBUF = 8


def _make_sc_lookup(batch: int, f_dim: int, emb: int):
    assert batch == _NW * _NTILES * _BT and emb == 2 * 8
    n_chunks = f_dim * _NTILES
    outer = n_chunks // _NBUF
    assert outer * _NBUF == n_chunks

    mesh = plsc.VectorSubcoreMesh(core_axis_name="c", subcore_axis_name="s")

    @functools.partial(
        pl.kernel,
        out_type=jax.ShapeDtypeStruct((f_dim, 2, batch // _BT, 8, _BT),
                                      jnp.float32),
        mesh=mesh,
        compiler_params=pltpu.CompilerParams(use_tc_tiling_on_sc=False,
                                             needs_layout_passes=False),
        scratch_types=[
            pltpu.VMEM((f_dim, _NTILES * _BT), jnp.int32),
            pltpu.VMEM((f_dim, _NTILES * _BT), jnp.float32),
            pltpu.VMEM((_NBUF, _BT, emb), jnp.float32),
            pltpu.VMEM((2, 2, _NTILES, 8, _BT), jnp.float32),
            pltpu.SemaphoreType.DMA((_NBUF,)),
            pltpu.SemaphoreType.DMA((2,)),
        ],
    )
    def lookup(idt_hbm, valt_hbm, table_hbm, out_hbm, idx_loc, val_loc,
               rows, ob, gsem, osem):
        wid = lax.axis_index("s") * _NUM_CORES + lax.axis_index("c")
        bcol = wid * (_NTILES * _BT)
        pltpu.sync_copy(idt_hbm.at[:, pl.ds(bcol, _NTILES * _BT)], idx_loc)
        pltpu.sync_copy(valt_hbm.at[:, pl.ds(bcol, _NTILES * _BT)], val_loc)

        def gather_desc(f, t, s):
            return pltpu.make_async_copy(
                table_hbm.at[idx_loc.at[f, pl.ds(t * _BT, _BT)]],
                rows.at[s], gsem.at[s])

        def out_desc(f, eb, s):
            return pltpu.make_async_copy(
                ob.at[s, eb],
                out_hbm.at[f, eb, pl.ds(wid * _NTILES, _NTILES)],
                osem.at[s])

        for k in range(_NBUF):
            gather_desc(k // _NTILES, k % _NTILES, k).start()

        row_idx = [lax.iota(jnp.int32, _LANES) + c * _LANES
                   for c in range(_BT // _LANES)]

        def outer_body(g, carry):
            for k in range(_NBUF):
                f = g * (_NBUF // _NTILES) + k // _NTILES
                t = k % _NTILES
                fs = k // _NTILES
                gather_desc(f, t, k).wait()

                if t == 0:
                    @pl.when(g > 0)
                    def _wait_prev_out():
                        out_desc(f, 0, fs).wait()
                        out_desc(f, 1, fs).wait()

                val_vecs = [val_loc[f, pl.ds(t * _BT + c * _LANES, _LANES)]
                            for c in range(_BT // _LANES)]
                for eb in range(2):
                    for es in range(8):
                        col = jnp.full((_LANES,), eb * 8 + es, jnp.int32)
                        for c in range(_BT // _LANES):
                            v = plsc.load_gather(rows.at[k],
                                                 [row_idx[c], col])
                            ob[fs, eb, t, es, pl.ds(c * _LANES, _LANES)] = (
                                v * val_vecs[c])
                if t == _NTILES - 1:
                    out_desc(f, 0, fs).start()
                    out_desc(f, 1, fs).start()

                @pl.when(g < outer - 1)
                def _next_gather():
                    gather_desc(f + _NBUF // _NTILES, t, k).start()
            return carry

        lax.fori_loop(0, outer, outer_body, 0)
        for fs in range(2):
            out_desc(f_dim - 1, 0, fs).wait()
            out_desc(f_dim - 1, 1, fs).wait()

    return lookup


def kernel(id, value, table):
    b, f = id.shape
    _, emb = table.shape
    idt = id.T.astype(jnp.int32)
    valt = value.T
    x = _make_sc_lookup(b, f, emb)(idt, valt, table)
    return x.transpose(2, 4, 0, 1, 3).reshape(b, f, emb)


# speedup vs baseline: 1.0137x; 1.0137x over previous
import functools

import jax
import jax.numpy as jnp
from jax import lax
from jax.experimental import pallas as pl
from jax.experimental.pallas import tpu as pltpu
from jax.experimental.pallas import tpu_sc as plsc

_NUM_CORES = 2
_NUM_SUBCORES = 16
_NW = _NUM_CORES * _NUM_SUBCORES
_LANES = 16

_BT = 128
_NTILES = 4
_NBUF = 8


def _make_sc_lookup(batch: int, f_dim: int, emb: int):
    assert batch == _NW * _NTILES * _BT and emb == 2 * 8
    n_chunks = f_dim * _NTILES
    outer = n_chunks // _NBUF
    assert outer * _NBUF == n_chunks

    mesh = plsc.VectorSubcoreMesh(core_axis_name="c", subcore_axis_name="s")

    @functools.partial(
        pl.kernel,
        out_type=jax.ShapeDtypeStruct((f_dim, 2, batch // _BT, 8, _BT),
                                      jnp.float32),
        mesh=mesh,
        compiler_params=pltpu.CompilerParams(use_tc_tiling_on_sc=False,
                                             needs_layout_passes=False),
        scratch_types=[
            pltpu.VMEM((f_dim, _NTILES * _BT), jnp.int32),
            pltpu.VMEM((f_dim, _NTILES * _BT), jnp.float32),
            pltpu.VMEM((_NBUF, _BT, emb), jnp.float32),
            pltpu.VMEM((_NBUF, 2, 8, _BT), jnp.float32),
            pltpu.SemaphoreType.DMA((_NBUF,)),
            pltpu.SemaphoreType.DMA((_NBUF,)),
        ],
    )
    def lookup(idt_hbm, valt_hbm, table_hbm, out_hbm, idx_loc, val_loc,
               rows, ob, gsem, osem):
        wid = lax.axis_index("s") * _NUM_CORES + lax.axis_index("c")
        bcol = wid * (_NTILES * _BT)
        pltpu.sync_copy(idt_hbm.at[:, pl.ds(bcol, _NTILES * _BT)], idx_loc)
        pltpu.sync_copy(valt_hbm.at[:, pl.ds(bcol, _NTILES * _BT)], val_loc)

        def gather_desc(f, t, s):
            return pltpu.make_async_copy(
                table_hbm.at[idx_loc.at[f, pl.ds(t * _BT, _BT)]],
                rows.at[s], gsem.at[s])

        def out_desc(f, eb, t, s):
            return pltpu.make_async_copy(
                ob.at[s, eb], out_hbm.at[f, eb, wid * _NTILES + t],
                osem.at[s])

        for k in range(_NBUF):
            gather_desc(k // _NTILES, k % _NTILES, k).start()

        row_idx = [lax.iota(jnp.int32, _LANES) + c * _LANES
                   for c in range(_BT // _LANES)]

        def outer_body(g, carry):
            for k in range(_NBUF):
                f = g * (_NBUF // _NTILES) + k // _NTILES
                t = k % _NTILES
                gather_desc(f, t, k).wait()

                @pl.when(g > 0)
                def _wait_prev_out():
                    out_desc(f, 0, t, k).wait()
                    out_desc(f, 1, t, k).wait()

                val_vecs = [val_loc[f, pl.ds(t * _BT + c * _LANES, _LANES)]
                            for c in range(_BT // _LANES)]
                for eb in range(2):
                    for es in range(8):
                        col = jnp.full((_LANES,), eb * 8 + es, jnp.int32)
                        for c in range(_BT // _LANES):
                            v = plsc.load_gather(rows.at[k],
                                                 [row_idx[c], col])
                            ob[k, eb, es, pl.ds(c * _LANES, _LANES)] = (
                                v * val_vecs[c])
                out_desc(f, 0, t, k).start()
                out_desc(f, 1, t, k).start()

                @pl.when(g < outer - 1)
                def _next_gather():
                    gather_desc(f + _NBUF // _NTILES, t, k).start()
            return carry

        lax.fori_loop(0, outer, outer_body, 0)
        for k in range(_NBUF):
            out_desc(f_dim - 1, 0, k % _NTILES, k).wait()
            out_desc(f_dim - 1, 1, k % _NTILES, k).wait()

    return lookup


def kernel(id, value, table):
    b, f = id.shape
    _, emb = table.shape
    idt = id.T.astype(jnp.int32)
    valt = value.T
    x = _make_sc_lookup(b, f, emb)(idt, valt, table)
    return x.transpose(2, 4, 0, 1, 3).reshape(b, f, emb)
